# baseline (speedup 1.0000x reference)
import functools

import jax
import jax.numpy as jnp
from jax import lax
from jax.experimental import pallas as pl
from jax.experimental.pallas import tpu as pltpu
from jax.experimental.pallas import tpu_sc as plsc

N = 10000
E = 160000
FN = 256
FE = 16
FG = 32
H = 512
FEO = 128
FNO = 256
G = 4

NB = 400
EBLK = 1600
NW = 32
NPT = 624
NREM = N - 16 * NPT
ZR = 24

_F32 = jnp.float32


def _pack(v):
    w = v.shape[1] // 2
    vb = v.astype(jnp.bfloat16)
    lo = lax.bitcast_convert_type(vb[:, :w], jnp.uint16).astype(jnp.uint32)
    hi = lax.bitcast_convert_type(vb[:, w:], jnp.uint16).astype(jnp.uint32)
    return lax.bitcast_convert_type(lo | (hi << 16), _F32)


def _unpack(p):
    u = lax.bitcast_convert_type(p, jnp.uint32)
    lo = lax.bitcast_convert_type(u << 16, _F32)
    hi = lax.bitcast_convert_type(u & jnp.uint32(0xFFFF0000), _F32)
    return lo, hi


def _pre_body(x_ref, bf_ref, g_ref, w_ref, b0_ref, xa_ref, xb_ref):
    x = x_ref[...]
    wsrc = w_ref[0:FN, :]
    wu = w_ref[2 * FN + FE:, :]
    gu = jnp.dot(g_ref[...], wu, preferred_element_type=_F32)
    oh = (bf_ref[...] == lax.broadcasted_iota(jnp.int32, (1, G), 1).astype(_F32)).astype(_F32)
    xa_ref[...] = _pack(jnp.dot(x, wsrc, preferred_element_type=_F32)
                        + jnp.dot(oh, gu, preferred_element_type=_F32)
                        + b0_ref[...])
    xb_ref[...] = _pack(x)


def _edge_body(sa_ref, sb_ref, attr_ref, wat_ref, wtg_ref, b1_ref, w1_ref,
               w2_ref, b2_ref, eg_ref, ebeta_ref, out_ref):
    a_lo, a_hi = _unpack(sa_ref[...])
    b_lo, b_hi = _unpack(sb_ref[...])
    tw = (jnp.dot(b_lo.astype(jnp.bfloat16), wtg_ref[0:FN // 2, :],
                  preferred_element_type=_F32)
          + jnp.dot(b_hi.astype(jnp.bfloat16), wtg_ref[FN // 2:, :],
                    preferred_element_type=_F32))
    aw = jnp.dot(attr_ref[...], wat_ref[...], preferred_element_type=_F32)
    h0_lo = jnp.maximum(a_lo + tw[:, :H // 2] + aw[:, :H // 2],
                        0.0).astype(jnp.bfloat16)
    h0_hi = jnp.maximum(a_hi + tw[:, H // 2:] + aw[:, H // 2:],
                        0.0).astype(jnp.bfloat16)
    h1 = jnp.maximum(
        jnp.dot(h0_lo, w1_ref[0:H // 2, :], preferred_element_type=_F32)
        + jnp.dot(h0_hi, w1_ref[H // 2:, :], preferred_element_type=_F32)
        + b1_ref[...],
        0.0)
    eh = jnp.dot(h1.astype(jnp.bfloat16), w2_ref[...],
                 preferred_element_type=_F32) + b2_ref[...]
    mu = jnp.mean(eh, axis=1, keepdims=True)
    var = jnp.mean((eh - mu) ** 2, axis=1, keepdims=True)
    out_ref[...] = (eh - mu) * lax.rsqrt(var + 1e-5) * eg_ref[...] + ebeta_ref[...]


def _node_body(x_ref, agg_ref, bf_ref, g_ref, w0_ref, b0_ref, w1_ref, b1_ref,
               w2_ref, b2_ref, ng_ref, nbeta_ref, out_ref):
    agg = agg_ref[0] + agg_ref[1]
    w0x = w0_ref[0:FN, :]
    w0a = w0_ref[FN:FN + FEO, :]
    w0u = w0_ref[FN + FEO:, :]
    gu = jnp.dot(g_ref[...], w0u, preferred_element_type=_F32)
    oh = (bf_ref[...] == lax.broadcasted_iota(jnp.int32, (1, G), 1).astype(_F32)).astype(_F32)
    h = jnp.maximum(
        jnp.dot(x_ref[...], w0x, preferred_element_type=_F32)
        + jnp.dot(agg, w0a, preferred_element_type=_F32)
        + jnp.dot(oh, gu, preferred_element_type=_F32)
        + b0_ref[...],
        0.0)
    h = jnp.maximum(
        jnp.dot(h, w1_ref[...], preferred_element_type=_F32) + b1_ref[...],
        0.0)
    nh = jnp.dot(h, w2_ref[...], preferred_element_type=_F32) + b2_ref[...]
    mu = jnp.mean(nh, axis=1, keepdims=True)
    var = jnp.mean((nh - mu) ** 2, axis=1, keepdims=True)
    out_ref[...] = (nh - mu) * lax.rsqrt(var + 1e-5) * ng_ref[...] + nbeta_ref[...]


def _full(shape):
    return pl.BlockSpec(shape, lambda i: tuple(0 for _ in shape))


def _precompute(x, bf, g, w, b0):
    return pl.pallas_call(
        _pre_body,
        grid=(N // NB,),
        in_specs=[
            pl.BlockSpec((NB, FN), lambda i: (i, 0)),
            pl.BlockSpec((NB, 1), lambda i: (i, 0)),
            _full((G, FG)),
            _full((2 * FN + FE + FG, H)),
            _full((1, H)),
        ],
        out_specs=[pl.BlockSpec((NB, H // 2), lambda i: (i, 0)),
                   pl.BlockSpec((NB, FN // 2), lambda i: (i, 0))],
        out_shape=[jax.ShapeDtypeStruct((N, H // 2), _F32),
                   jax.ShapeDtypeStruct((N, FN // 2), _F32)],
    )(x, bf, g, w, b0)


def _edge_mlp(sa, sb, attr, wat, wtg, w1, b1, w2, b2, eg, ebeta):
    return pl.pallas_call(
        _edge_body,
        grid=(sa.shape[0] // EBLK,),
        in_specs=[
            pl.BlockSpec((EBLK, H // 2), lambda i: (i, 0)),
            pl.BlockSpec((EBLK, FN // 2), lambda i: (i, 0)),
            pl.BlockSpec((EBLK, FE), lambda i: (i, 0)),
            _full((FE, H)),
            _full((FN, H)),
            _full((1, H)),
            _full((H, H)),
            _full((H, FEO)),
            _full((1, FEO)),
            _full((1, FEO)),
            _full((1, FEO)),
        ],
        out_specs=pl.BlockSpec((EBLK, FEO), lambda i: (i, 0)),
        out_shape=jax.ShapeDtypeStruct((sa.shape[0], FEO), _F32),
    )(sa, sb, attr, wat, wtg, b1, w1, w2, b2, eg, ebeta)


def _node_mlp(x, agg, bf, g, w0, b0, w1, b1, w2, b2, ng, nbeta):
    return pl.pallas_call(
        _node_body,
        grid=(N // NB,),
        in_specs=[
            pl.BlockSpec((NB, FN), lambda i: (i, 0)),
            pl.BlockSpec((2, NB, FEO), lambda i: (0, i, 0)),
            pl.BlockSpec((NB, 1), lambda i: (i, 0)),
            _full((G, FG)),
            _full((FN + FEO + FG, H)),
            _full((1, H)),
            _full((H, H)),
            _full((1, H)),
            _full((H, FNO)),
            _full((1, FNO)),
            _full((1, FNO)),
            _full((1, FNO)),
        ],
        out_specs=pl.BlockSpec((NB, FNO), lambda i: (i, 0)),
        out_shape=jax.ShapeDtypeStruct((N, FNO), _F32),
    )(x, agg, bf, g, w0, b0, w1, b1, w2, b2, ng, nbeta)


_MESH = plsc.VectorSubcoreMesh(core_axis_name="c", subcore_axis_name="s")


G_C = 64
G_NCH = E // G_C
G_NQMAX = 79
_NSLOT = 3


@functools.partial(
    pl.kernel, mesh=_MESH,
    out_type=(jax.ShapeDtypeStruct((E, H // 2), _F32),
              jax.ShapeDtypeStruct((E, FN // 2), _F32)),
    scratch_types=[
        pltpu.VMEM((G_NQMAX, 1, G_C), jnp.int32),
        pltpu.VMEM((G_NQMAX, 1, G_C), jnp.int32),
    ] + [pltpu.VMEM((G_C, H // 2), _F32)] * _NSLOT
      + [pltpu.VMEM((G_C, FN // 2), _F32)] * _NSLOT
      + [pltpu.SemaphoreType.DMA] * (4 * _NSLOT))
def _gather(xa_hbm, xb_hbm, rowi_hbm, coli_hbm, sa_hbm, sb_hbm,
            idxa, idxb, *bufsem):
    bufa = bufsem[0:_NSLOT]
    bufb = bufsem[_NSLOT:2 * _NSLOT]
    ga = bufsem[2 * _NSLOT:3 * _NSLOT]
    gb = bufsem[3 * _NSLOT:4 * _NSLOT]
    wa = bufsem[4 * _NSLOT:5 * _NSLOT]
    wb = bufsem[5 * _NSLOT:6 * _NSLOT]
    wid = lax.axis_index("s") * 2 + lax.axis_index("c")
    qa = (G_NCH * wid) // NW
    qb = (G_NCH * (wid + 1)) // NW
    nq = qb - qa
    pltpu.sync_copy(rowi_hbm.at[pl.ds(qa, G_NQMAX)], idxa)
    pltpu.sync_copy(coli_hbm.at[pl.ds(qa, G_NQMAX)], idxb)

    def issue_gather(t, k):
        pltpu.async_copy(xa_hbm.at[idxa.at[t, 0]], bufa[k], ga[k])
        pltpu.async_copy(xb_hbm.at[idxb.at[t, 0]], bufb[k], gb[k])

    def drain_gather(t, k):
        pltpu.make_async_copy(xa_hbm.at[idxa.at[t, 0]], bufa[k], ga[k]).wait()
        pltpu.make_async_copy(xb_hbm.at[idxb.at[t, 0]], bufb[k], gb[k]).wait()

    def issue_write(t, k):
        off = (qa + t) * G_C
        pltpu.async_copy(bufa[k], sa_hbm.at[pl.ds(off, G_C)], wa[k])
        pltpu.async_copy(bufb[k], sb_hbm.at[pl.ds(off, G_C)], wb[k])

    def wait_write(t, k):
        off = (qa + t) * G_C
        pltpu.make_async_copy(bufa[k], sa_hbm.at[pl.ds(off, G_C)], wa[k]).wait()
        pltpu.make_async_copy(bufb[k], sb_hbm.at[pl.ds(off, G_C)], wb[k]).wait()

    for k in range(_NSLOT):
        issue_gather(k, k)

    def body(t, carry):
        for k in range(_NSLOT):

            @pl.when(t % _NSLOT == k)
            def _slot():
                drain_gather(t, k)
                issue_write(t, k)

                @pl.when(t + _NSLOT < nq)
                def _next():
                    wait_write(t, k)
                    issue_gather(t + _NSLOT, k)

        return carry

    lax.fori_loop(0, nq, body, 0)
    @pl.when(nq == 78)
    def _tail78():
        wait_write(75, 0)
        wait_write(76, 1)
        wait_write(77, 2)

    @pl.when(nq == 79)
    def _tail79():
        wait_write(76, 1)
        wait_write(77, 2)
        wait_write(78, 0)


SC_C = 64
SC_NCH = E // SC_C


@functools.partial(
    pl.kernel, mesh=_MESH,
    out_type=jax.ShapeDtypeStruct((2, N, FEO), _F32),
    scratch_types=[
        pltpu.VMEM((1, SC_C), jnp.int32),
        pltpu.VMEM((1, SC_C), jnp.int32),
        pltpu.VMEM((SC_C, FEO), _F32),
        pltpu.VMEM((SC_C, FEO), _F32),
        pltpu.VMEM((ZR, FEO), _F32),
        pltpu.VMEM_SHARED((N, FEO), _F32),
        pltpu.SemaphoreType.DMA,
        pltpu.SemaphoreType.DMA,
        pltpu.SemaphoreType.DMA,
        pltpu.SemaphoreType.DMA,
    ])
def _scatter(eo_hbm, coli_hbm, agg_hbm, idx0, idx1, buf0, buf1, zbuf, shared,
             si0, si1, sd0, sd1):
    cid = lax.axis_index("c")
    sid = lax.axis_index("s")
    wid = sid * 2 + cid

    def zb(r, carry):
        for k in range(FEO // 16):
            zbuf[r, pl.ds(k * 16, 16)] = jnp.zeros((16,), _F32)
        return carry

    lax.fori_loop(0, ZR, zb, 0)

    def zs(j, carry):
        pltpu.sync_copy(zbuf, shared.at[pl.ds(sid * NPT + j * ZR, ZR)])
        return carry

    lax.fori_loop(0, NPT // ZR, zs, 0)

    @pl.when(sid == 0)
    def _zero_tail():
        pltpu.sync_copy(zbuf.at[pl.ds(0, NREM)],
                        shared.at[pl.ds(16 * NPT, NREM)])

    plsc.subcore_barrier()

    qa = (SC_NCH * wid) // NW
    qb = (SC_NCH * (wid + 1)) // NW
    nq = qb - qa

    def rd(t, ib, b, smi, smd):
        q = qa + t
        pltpu.async_copy(coli_hbm.at[q], ib, smi)
        pltpu.async_copy(eo_hbm.at[pl.ds(q * SC_C, SC_C)], b, smd)

    def add(t, ib, b, smi, smd):
        q = qa + t
        pltpu.make_async_copy(coli_hbm.at[q], ib, smi).wait()
        pltpu.make_async_copy(
            eo_hbm.at[pl.ds(q * SC_C, SC_C)], b, smd).wait()
        pltpu.sync_copy(b, shared.at[ib.at[0]], add=True)

    rd(0, idx0, buf0, si0, sd0)

    def body(t, carry):
        par = t % 2

        @pl.when(t + 1 < nq)
        def _prefetch():
            @pl.when(par == 0)
            def _p0():
                rd(t + 1, idx1, buf1, si1, sd1)

            @pl.when(par == 1)
            def _p1():
                rd(t + 1, idx0, buf0, si0, sd0)

        @pl.when(par == 0)
        def _a0():
            add(t, idx0, buf0, si0, sd0)

        @pl.when(par == 1)
        def _a1():
            add(t, idx1, buf1, si1, sd1)

        return carry

    lax.fori_loop(0, nq, body, 0)
    plsc.subcore_barrier()
    pltpu.sync_copy(shared.at[pl.ds(sid * NPT, NPT)],
                    agg_hbm.at[cid, pl.ds(sid * NPT, NPT)])

    @pl.when(sid == 0)
    def _copy_tail():
        pltpu.sync_copy(shared.at[pl.ds(16 * NPT, NREM)],
                        agg_hbm.at[cid, pl.ds(16 * NPT, NREM)])


def kernel(x, edge_index, edge_attr, global_features, batch,
           eW0, eb0, eW1, eb1, eW2, eb2, eg, ebeta,
           nW0, nb0, nW1, nb1, nW2, nb2, ng, nbeta):
    row = edge_index[0].astype(jnp.int32)
    col = edge_index[1].astype(jnp.int32)
    bf = batch.astype(_F32).reshape(N, 1)

    xa, xb = _precompute(x, bf, global_features, eW0, eb0.reshape(1, H))
    sa, sb = _gather(xa, xb, row.reshape(G_NCH, 1, G_C),
                     col.reshape(G_NCH, 1, G_C))
    edge_out = _edge_mlp(sa, sb, edge_attr, eW0[2 * FN:2 * FN + FE],
                         eW0[FN:2 * FN].astype(jnp.bfloat16),
                         eW1.astype(jnp.bfloat16), eb1.reshape(1, H),
                         eW2.astype(jnp.bfloat16), eb2.reshape(1, FEO),
                         eg.reshape(1, FEO), ebeta.reshape(1, FEO))
    agg = _scatter(edge_out, col.reshape(SC_NCH, 1, SC_C))
    node_out = _node_mlp(x, agg, bf, global_features,
                         nW0, nb0.reshape(1, H), nW1, nb1.reshape(1, H),
                         nW2, nb2.reshape(1, FNO), ng.reshape(1, FNO),
                         nbeta.reshape(1, FNO))
    return node_out, edge_out, global_features


# speedup vs baseline: 1.0709x; 1.0709x over previous
import functools

import jax
import jax.numpy as jnp
from jax import lax
from jax.experimental import pallas as pl
from jax.experimental.pallas import tpu as pltpu
from jax.experimental.pallas import tpu_sc as plsc

N = 10000
E = 160000
FN = 256
FE = 16
FG = 32
H = 512
FEO = 128
FNO = 256
G = 4

NB = 400
EBLK = 1600
NW = 32
NPT = 624
NREM = N - 16 * NPT
ZR = 24

_F32 = jnp.float32


def _pack(v):
    vb = v.astype(jnp.bfloat16)
    lo = lax.bitcast_convert_type(vb[:, :H // 2], jnp.uint16).astype(jnp.uint32)
    hi = lax.bitcast_convert_type(vb[:, H // 2:], jnp.uint16).astype(jnp.uint32)
    return lax.bitcast_convert_type(lo | (hi << 16), _F32)


def _unpack(p):
    u = lax.bitcast_convert_type(p, jnp.uint32)
    lo = lax.bitcast_convert_type(u << 16, _F32)
    hi = lax.bitcast_convert_type(u & jnp.uint32(0xFFFF0000), _F32)
    return lo, hi


def _pre_body(x_ref, bf_ref, g_ref, w_ref, b0_ref, xa_ref, xb_ref):
    x = x_ref[...]
    wsrc = w_ref[0:FN, :]
    wtgt = w_ref[FN:2 * FN, :]
    wu = w_ref[2 * FN + FE:, :]
    gu = jnp.dot(g_ref[...], wu, preferred_element_type=_F32)
    oh = (bf_ref[...] == lax.broadcasted_iota(jnp.int32, (1, G), 1).astype(_F32)).astype(_F32)
    xa_ref[...] = _pack(jnp.dot(x, wsrc, preferred_element_type=_F32)
                        + jnp.dot(oh, gu, preferred_element_type=_F32)
                        + b0_ref[...])
    xb_ref[...] = _pack(jnp.dot(x, wtgt, preferred_element_type=_F32))


def _edge_body(sa_ref, sb_ref, attr_ref, wat_ref, b1_ref, w1_ref, w2_ref,
               b2_ref, eg_ref, ebeta_ref, out_ref):
    a_lo, a_hi = _unpack(sa_ref[...])
    b_lo, b_hi = _unpack(sb_ref[...])
    aw = jnp.dot(attr_ref[...], wat_ref[...], preferred_element_type=_F32)
    h0_lo = jnp.maximum(a_lo + b_lo + aw[:, :H // 2], 0.0).astype(jnp.bfloat16)
    h0_hi = jnp.maximum(a_hi + b_hi + aw[:, H // 2:], 0.0).astype(jnp.bfloat16)
    h1 = jnp.maximum(
        jnp.dot(h0_lo, w1_ref[0:H // 2, :], preferred_element_type=_F32)
        + jnp.dot(h0_hi, w1_ref[H // 2:, :], preferred_element_type=_F32)
        + b1_ref[...],
        0.0)
    eh = jnp.dot(h1.astype(jnp.bfloat16), w2_ref[...],
                 preferred_element_type=_F32) + b2_ref[...]
    mu = jnp.mean(eh, axis=1, keepdims=True)
    var = jnp.mean((eh - mu) ** 2, axis=1, keepdims=True)
    out_ref[...] = (eh - mu) * lax.rsqrt(var + 1e-5) * eg_ref[...] + ebeta_ref[...]


def _node_body(x_ref, agg_ref, bf_ref, g_ref, w0_ref, b0_ref, w1_ref, b1_ref,
               w2_ref, b2_ref, ng_ref, nbeta_ref, out_ref):
    agg = agg_ref[0] + agg_ref[1]
    w0x = w0_ref[0:FN, :]
    w0a = w0_ref[FN:FN + FEO, :]
    w0u = w0_ref[FN + FEO:, :]
    gu = jnp.dot(g_ref[...], w0u, preferred_element_type=_F32)
    oh = (bf_ref[...] == lax.broadcasted_iota(jnp.int32, (1, G), 1).astype(_F32)).astype(_F32)
    h = jnp.maximum(
        jnp.dot(x_ref[...], w0x, preferred_element_type=_F32)
        + jnp.dot(agg, w0a, preferred_element_type=_F32)
        + jnp.dot(oh, gu, preferred_element_type=_F32)
        + b0_ref[...],
        0.0)
    h = jnp.maximum(
        jnp.dot(h, w1_ref[...], preferred_element_type=_F32) + b1_ref[...],
        0.0)
    nh = jnp.dot(h, w2_ref[...], preferred_element_type=_F32) + b2_ref[...]
    mu = jnp.mean(nh, axis=1, keepdims=True)
    var = jnp.mean((nh - mu) ** 2, axis=1, keepdims=True)
    out_ref[...] = (nh - mu) * lax.rsqrt(var + 1e-5) * ng_ref[...] + nbeta_ref[...]


def _full(shape):
    return pl.BlockSpec(shape, lambda i: tuple(0 for _ in shape))


def _precompute(x, bf, g, w, b0):
    return pl.pallas_call(
        _pre_body,
        grid=(N // NB,),
        in_specs=[
            pl.BlockSpec((NB, FN), lambda i: (i, 0)),
            pl.BlockSpec((NB, 1), lambda i: (i, 0)),
            _full((G, FG)),
            _full((2 * FN + FE + FG, H)),
            _full((1, H)),
        ],
        out_specs=[pl.BlockSpec((NB, H // 2), lambda i: (i, 0)),
                   pl.BlockSpec((NB, H // 2), lambda i: (i, 0))],
        out_shape=[jax.ShapeDtypeStruct((N, H // 2), _F32),
                   jax.ShapeDtypeStruct((N, H // 2), _F32)],
    )(x, bf, g, w, b0)


def _edge_mlp(sa, sb, attr, wat, w1, b1, w2, b2, eg, ebeta):
    return pl.pallas_call(
        _edge_body,
        grid=(sa.shape[0] // EBLK,),
        in_specs=[
            pl.BlockSpec((EBLK, H // 2), lambda i: (i, 0)),
            pl.BlockSpec((EBLK, H // 2), lambda i: (i, 0)),
            pl.BlockSpec((EBLK, FE), lambda i: (i, 0)),
            _full((FE, H)),
            _full((1, H)),
            _full((H, H)),
            _full((H, FEO)),
            _full((1, FEO)),
            _full((1, FEO)),
            _full((1, FEO)),
        ],
        out_specs=pl.BlockSpec((EBLK, FEO), lambda i: (i, 0)),
        out_shape=jax.ShapeDtypeStruct((sa.shape[0], FEO), _F32),
    )(sa, sb, attr, wat, b1, w1, w2, b2, eg, ebeta)


def _node_mlp(x, agg, bf, g, w0, b0, w1, b1, w2, b2, ng, nbeta):
    return pl.pallas_call(
        _node_body,
        grid=(N // NB,),
        in_specs=[
            pl.BlockSpec((NB, FN), lambda i: (i, 0)),
            pl.BlockSpec((2, NB, FEO), lambda i: (0, i, 0)),
            pl.BlockSpec((NB, 1), lambda i: (i, 0)),
            _full((G, FG)),
            _full((FN + FEO + FG, H)),
            _full((1, H)),
            _full((H, H)),
            _full((1, H)),
            _full((H, FNO)),
            _full((1, FNO)),
            _full((1, FNO)),
            _full((1, FNO)),
        ],
        out_specs=pl.BlockSpec((NB, FNO), lambda i: (i, 0)),
        out_shape=jax.ShapeDtypeStruct((N, FNO), _F32),
    )(x, agg, bf, g, w0, b0, w1, b1, w2, b2, ng, nbeta)


_MESH = plsc.VectorSubcoreMesh(core_axis_name="c", subcore_axis_name="s")


G_C = 64
G_NCH = E // G_C
G_NQMAX = 79
_NSLOT = 3


@functools.partial(
    pl.kernel, mesh=_MESH,
    out_type=(jax.ShapeDtypeStruct((E, H // 2), _F32),
              jax.ShapeDtypeStruct((E, H // 2), _F32)),
    scratch_types=[
        pltpu.VMEM((G_NQMAX, 1, G_C), jnp.int32),
        pltpu.VMEM((G_NQMAX, 1, G_C), jnp.int32),
    ] + [pltpu.VMEM((G_C, H // 2), _F32)] * (2 * _NSLOT)
      + [pltpu.SemaphoreType.DMA] * (4 * _NSLOT))
def _gather(xa_hbm, xb_hbm, rowi_hbm, coli_hbm, sa_hbm, sb_hbm,
            idxa, idxb, *bufsem):
    bufa = bufsem[0:_NSLOT]
    bufb = bufsem[_NSLOT:2 * _NSLOT]
    ga = bufsem[2 * _NSLOT:3 * _NSLOT]
    gb = bufsem[3 * _NSLOT:4 * _NSLOT]
    wa = bufsem[4 * _NSLOT:5 * _NSLOT]
    wb = bufsem[5 * _NSLOT:6 * _NSLOT]
    wid = lax.axis_index("s") * 2 + lax.axis_index("c")
    qa = (G_NCH * wid) // NW
    qb = (G_NCH * (wid + 1)) // NW
    nq = qb - qa
    pltpu.sync_copy(rowi_hbm.at[pl.ds(qa, G_NQMAX)], idxa)
    pltpu.sync_copy(coli_hbm.at[pl.ds(qa, G_NQMAX)], idxb)

    def issue_gather(t, k):
        pltpu.async_copy(xa_hbm.at[idxa.at[t, 0]], bufa[k], ga[k])
        pltpu.async_copy(xb_hbm.at[idxb.at[t, 0]], bufb[k], gb[k])

    def drain_gather(t, k):
        pltpu.make_async_copy(xa_hbm.at[idxa.at[t, 0]], bufa[k], ga[k]).wait()
        pltpu.make_async_copy(xb_hbm.at[idxb.at[t, 0]], bufb[k], gb[k]).wait()

    def issue_write(t, k):
        off = (qa + t) * G_C
        pltpu.async_copy(bufa[k], sa_hbm.at[pl.ds(off, G_C)], wa[k])
        pltpu.async_copy(bufb[k], sb_hbm.at[pl.ds(off, G_C)], wb[k])

    def wait_write(t, k):
        off = (qa + t) * G_C
        pltpu.make_async_copy(bufa[k], sa_hbm.at[pl.ds(off, G_C)], wa[k]).wait()
        pltpu.make_async_copy(bufb[k], sb_hbm.at[pl.ds(off, G_C)], wb[k]).wait()

    for k in range(_NSLOT):
        issue_gather(k, k)

    def body(t, carry):
        for k in range(_NSLOT):

            @pl.when(t % _NSLOT == k)
            def _slot():
                drain_gather(t, k)
                issue_write(t, k)

                @pl.when(t + _NSLOT < nq)
                def _next():
                    wait_write(t, k)
                    issue_gather(t + _NSLOT, k)

        return carry

    lax.fori_loop(0, nq, body, 0)
    @pl.when(nq == 78)
    def _tail78():
        wait_write(75, 0)
        wait_write(76, 1)
        wait_write(77, 2)

    @pl.when(nq == 79)
    def _tail79():
        wait_write(76, 1)
        wait_write(77, 2)
        wait_write(78, 0)


SC_C = 64
SC_NCH = E // SC_C


@functools.partial(
    pl.kernel, mesh=_MESH,
    out_type=jax.ShapeDtypeStruct((2, N, FEO), _F32),
    scratch_types=[
        pltpu.VMEM((1, SC_C), jnp.int32),
        pltpu.VMEM((1, SC_C), jnp.int32),
        pltpu.VMEM((SC_C, FEO), _F32),
        pltpu.VMEM((SC_C, FEO), _F32),
        pltpu.VMEM((ZR, FEO), _F32),
        pltpu.VMEM_SHARED((N, FEO), _F32),
        pltpu.SemaphoreType.DMA,
        pltpu.SemaphoreType.DMA,
        pltpu.SemaphoreType.DMA,
        pltpu.SemaphoreType.DMA,
    ])
def _scatter(eo_hbm, coli_hbm, agg_hbm, idx0, idx1, buf0, buf1, zbuf, shared,
             si0, si1, sd0, sd1):
    cid = lax.axis_index("c")
    sid = lax.axis_index("s")
    wid = sid * 2 + cid

    def zb(r, carry):
        for k in range(FEO // 16):
            zbuf[r, pl.ds(k * 16, 16)] = jnp.zeros((16,), _F32)
        return carry

    lax.fori_loop(0, ZR, zb, 0)

    def zs(j, carry):
        pltpu.sync_copy(zbuf, shared.at[pl.ds(sid * NPT + j * ZR, ZR)])
        return carry

    lax.fori_loop(0, NPT // ZR, zs, 0)

    @pl.when(sid == 0)
    def _zero_tail():
        pltpu.sync_copy(zbuf.at[pl.ds(0, NREM)],
                        shared.at[pl.ds(16 * NPT, NREM)])

    plsc.subcore_barrier()

    qa = (SC_NCH * wid) // NW
    qb = (SC_NCH * (wid + 1)) // NW
    nq = qb - qa

    def rd(t, ib, b, smi, smd):
        q = qa + t
        pltpu.async_copy(coli_hbm.at[q], ib, smi)
        pltpu.async_copy(eo_hbm.at[pl.ds(q * SC_C, SC_C)], b, smd)

    def add(t, ib, b, smi, smd):
        q = qa + t
        pltpu.make_async_copy(coli_hbm.at[q], ib, smi).wait()
        pltpu.make_async_copy(
            eo_hbm.at[pl.ds(q * SC_C, SC_C)], b, smd).wait()
        pltpu.sync_copy(b, shared.at[ib.at[0]], add=True)

    rd(0, idx0, buf0, si0, sd0)

    def body(t, carry):
        par = t % 2

        @pl.when(t + 1 < nq)
        def _prefetch():
            @pl.when(par == 0)
            def _p0():
                rd(t + 1, idx1, buf1, si1, sd1)

            @pl.when(par == 1)
            def _p1():
                rd(t + 1, idx0, buf0, si0, sd0)

        @pl.when(par == 0)
        def _a0():
            add(t, idx0, buf0, si0, sd0)

        @pl.when(par == 1)
        def _a1():
            add(t, idx1, buf1, si1, sd1)

        return carry

    lax.fori_loop(0, nq, body, 0)
    plsc.subcore_barrier()
    pltpu.sync_copy(shared.at[pl.ds(sid * NPT, NPT)],
                    agg_hbm.at[cid, pl.ds(sid * NPT, NPT)])

    @pl.when(sid == 0)
    def _copy_tail():
        pltpu.sync_copy(shared.at[pl.ds(16 * NPT, NREM)],
                        agg_hbm.at[cid, pl.ds(16 * NPT, NREM)])


def kernel(x, edge_index, edge_attr, global_features, batch,
           eW0, eb0, eW1, eb1, eW2, eb2, eg, ebeta,
           nW0, nb0, nW1, nb1, nW2, nb2, ng, nbeta):
    row = edge_index[0].astype(jnp.int32)
    col = edge_index[1].astype(jnp.int32)
    bf = batch.astype(_F32).reshape(N, 1)

    xa, xb = _precompute(x, bf, global_features, eW0, eb0.reshape(1, H))
    sa, sb = _gather(xa, xb, row.reshape(G_NCH, 1, G_C),
                     col.reshape(G_NCH, 1, G_C))
    edge_out = _edge_mlp(sa, sb, edge_attr, eW0[2 * FN:2 * FN + FE],
                         eW1.astype(jnp.bfloat16), eb1.reshape(1, H),
                         eW2.astype(jnp.bfloat16), eb2.reshape(1, FEO),
                         eg.reshape(1, FEO), ebeta.reshape(1, FEO))
    agg = _scatter(edge_out, col.reshape(SC_NCH, 1, SC_C))
    node_out = _node_mlp(x, agg, bf, global_features,
                         nW0, nb0.reshape(1, H), nW1, nb1.reshape(1, H),
                         nW2, nb2.reshape(1, FNO), ng.reshape(1, FNO),
                         nbeta.reshape(1, FNO))
    return node_out, edge_out, global_features
